# baseline (speedup 1.0000x reference)
import functools

import jax
import jax.numpy as jnp
import numpy as np
from jax import lax
from jax.experimental import pallas as pl
from jax.experimental.pallas import tpu as pltpu
from jax.experimental.pallas import tpu_sc as plsc

B, P, D, E, L = 100, 99, 256, 160000, 3
N = B * (P + 1)
NP = 10240
NPH = NP // 2
EP = 163840
NC, NS = 2, 16
ET = EP // NS
EA = EP // (NC * NS)
CAP = 5632
CEC = 64
CCH = CAP // CEC
GC = 8
RW = NPH // NS
H2 = D // 2
RB = 1280

_mesh = plsc.VectorSubcoreMesh(
    core_axis_name="c", subcore_axis_name="s", num_cores=NC, num_subcores=NS)

_f32 = jnp.float32
_bf16 = jnp.bfloat16

_cp = np.empty((D,), np.int32)
for _j in range(D // 32):
    for _t in range(16):
        _cp[32 * _j + 2 * _t] = 32 * _j + _t
        _cp[32 * _j + 2 * _t + 1] = 32 * _j + 16 + _t
_COLPERM = _cp


@functools.partial(
    pl.kernel,
    out_type=jax.ShapeDtypeStruct((NC * NS, NP), _f32),
    mesh=_mesh,
    compiler_params=pltpu.CompilerParams(needs_layout_passes=False),
    scratch_types=[
        pltpu.VMEM((NP,), _f32),
        pltpu.VMEM((EA,), jnp.int32),
        pltpu.VMEM((EA,), _f32),
    ],
)
def _deg_kernel(dst_hbm, w_hbm, deg_hbm, acc_v, dstv, wv):
    cid = lax.axis_index("c")
    sid = lax.axis_index("s")
    wid = cid * NS + sid
    zero16 = lax.iota(jnp.int32, 16).astype(_f32) * 0.0

    def _z(i, _):
        acc_v[pl.ds(i * 16, 16)] = zero16
        return 0

    lax.fori_loop(0, NP // 16, _z, 0)
    base = wid * EA
    pltpu.sync_copy(dst_hbm.at[pl.ds(base, EA)], dstv)
    pltpu.sync_copy(w_hbm.at[pl.ds(base, EA)], wv)

    def _grp(g, _):
        sl = pl.ds(g * 16, 16)
        plsc.addupdate_scatter(acc_v, [dstv[sl]], wv[sl])
        return 0

    lax.fori_loop(0, EA // 16, _grp, 0)
    pltpu.sync_copy(acc_v, deg_hbm.at[wid])


@functools.partial(
    pl.kernel,
    out_type=(jax.ShapeDtypeStruct((NS * NC, CAP), jnp.int32),
              jax.ShapeDtypeStruct((NS * NC, CAP), jnp.int32),
              jax.ShapeDtypeStruct((NS * NC, CAP), _f32)),
    mesh=_mesh,
    compiler_params=pltpu.CompilerParams(needs_layout_passes=False),
    scratch_types=[
        pltpu.VMEM((NP,), _f32),
        pltpu.VMEM((ET,), jnp.int32),
        pltpu.VMEM((ET,), jnp.int32),
        pltpu.VMEM((ET,), _f32),
        pltpu.VMEM((CAP + 16,), jnp.int32),
        pltpu.VMEM((CAP + 16,), jnp.int32),
        pltpu.VMEM((CAP + 16,), _f32),
        pltpu.VMEM((CAP + 16,), jnp.int32),
        pltpu.VMEM((CAP + 16,), jnp.int32),
        pltpu.VMEM((CAP + 16,), _f32),
    ],
)
def _prep_kernel(src_hbm, dst_hbm, w_hbm, dinv_hbm,
                 srcp_hbm, dstp_hbm, nrmp_hbm,
                 dinv_v, srcv, dstv, wv, sA, dA, nA, sB, dB, nB):
    cid = lax.axis_index("c")
    sid = lax.axis_index("s")

    @pl.when(cid == 0)
    def _():
        izero = lax.iota(jnp.int32, 16) * 0
        zero16 = izero.astype(_f32)
        pltpu.sync_copy(dinv_hbm, dinv_v)
        base = sid * ET
        pltpu.sync_copy(src_hbm.at[pl.ds(base, ET)], srcv)
        pltpu.sync_copy(dst_hbm.at[pl.ds(base, ET)], dstv)
        pltpu.sync_copy(w_hbm.at[pl.ds(base, ET)], wv)

        def _z(i, _):
            sl = pl.ds(i * 16, 16)
            sA[sl] = izero
            dA[sl] = izero
            nA[sl] = zero16
            sB[sl] = izero
            dB[sl] = izero
            nB[sl] = zero16
            return 0

        lax.fori_loop(0, (CAP + 16) // 16, _z, 0)

        def _g(g, carry):
            cA, cB = carry
            sl = pl.ds(g * 16, 16)
            s16 = srcv[sl]
            d16 = dstv[sl]
            nrm = (wv[sl] * plsc.load_gather(dinv_v, [s16])
                   * plsc.load_gather(dinv_v, [d16]))
            m0 = d16 < NPH
            m1 = jnp.logical_not(m0)
            pc0 = plsc.cumsum(m0.astype(jnp.int32))
            pc1 = plsc.cumsum(m1.astype(jnp.int32))
            pos0 = cA + pc0 - 1
            pos1 = cB + pc1 - 1
            plsc.store_scatter(sA, [pos0], s16, mask=m0)
            plsc.store_scatter(dA, [pos0], d16, mask=m0)
            plsc.store_scatter(nA, [pos0], nrm, mask=m0)
            plsc.store_scatter(sB, [pos1], s16, mask=m1)
            plsc.store_scatter(dB, [pos1], d16 - NPH, mask=m1)
            plsc.store_scatter(nB, [pos1], nrm, mask=m1)
            k0 = plsc.all_reduce_population_count(m0)[0]
            return (cA + k0, cB + (16 - k0))

        lax.fori_loop(0, ET // 16, _g, (0, 0))
        pltpu.sync_copy(sA.at[pl.ds(0, CAP)], srcp_hbm.at[sid * NC])
        pltpu.sync_copy(dA.at[pl.ds(0, CAP)], dstp_hbm.at[sid * NC])
        pltpu.sync_copy(nA.at[pl.ds(0, CAP)], nrmp_hbm.at[sid * NC])
        pltpu.sync_copy(sB.at[pl.ds(0, CAP)], srcp_hbm.at[sid * NC + 1])
        pltpu.sync_copy(dB.at[pl.ds(0, CAP)], dstp_hbm.at[sid * NC + 1])
        pltpu.sync_copy(nB.at[pl.ds(0, CAP)], nrmp_hbm.at[sid * NC + 1])


@functools.partial(
    pl.kernel,
    out_type=jax.ShapeDtypeStruct((2 * NP, H2), _f32),
    mesh=_mesh,
    compiler_params=pltpu.CompilerParams(needs_layout_passes=False),
    scratch_types=[
        pltpu.VMEM((RW,), _f32),
        pltpu.VMEM((GC, CEC), jnp.int32),
        pltpu.VMEM((GC, CEC), jnp.int32),
        pltpu.VMEM((GC, CEC), jnp.int32),
        pltpu.VMEM((GC, CEC), _f32),
        pltpu.VMEM((CEC, H2), jnp.int32),
        pltpu.VMEM((CEC, H2), jnp.int32),
        pltpu.VMEM((CEC, H2), _f32),
        pltpu.VMEM((CEC, H2), _f32),
        pltpu.VMEM_SHARED((2 * NPH, H2), _f32),
        pltpu.SemaphoreType.DMA,
        pltpu.SemaphoreType.DMA,
        pltpu.SemaphoreType.DMA,
        pltpu.SemaphoreType.DMA,
    ],
)
def _conv_kernel(y_hbm, src_hbm, dst_hbm, norm_hbm, dinv2_hbm, acc_hbm,
                 dinv2_v, src2, dst2, dst2b, nrm2, bb0, bb1, sbA, sbB,
                 acc_sh, gs0, gs1, ssA, ssB):
    cid = lax.axis_index("c")
    sid = lax.axis_index("s")
    izero = lax.iota(jnp.int32, 16) * 0
    bbufs = [bb0, bb1]
    gsems = [gs0, gs1]
    nbase = cid * NPH + sid * RW

    pltpu.sync_copy(dinv2_hbm.at[pl.ds(nbase, RW)], dinv2_v)

    zero16 = izero.astype(_f32)

    def _zv(i, _):
        sbA[i // (H2 // 16), pl.ds((i % (H2 // 16)) * 16, 16)] = zero16
        return 0

    lax.fori_loop(0, CEC * (H2 // 16), _zv, 0)
    for b in range(RW // CEC):
        pltpu.sync_copy(sbA, acc_sh.at[pl.ds(sid * RW + b * CEC, CEC)])
        pltpu.sync_copy(sbA, acc_sh.at[pl.ds(NPH + sid * RW + b * CEC, CEC)])
    plsc.subcore_barrier()

    def _scale(bbuf, k):
        def _body(e, _):
            wev = plsc.load_gather(nrm2.at[k], [izero + e])
            for j in range(D // 32):
                ab = plsc.bitcast(bbuf[e, pl.ds(j * 16, 16)], _bf16)
                lo, hi = plsc.unpack(ab, format=plsc.PackFormat.INTERLEAVED)
                tgt = sbA if j < 4 else sbB
                jj = j % 4
                tgt[e, pl.ds(jj * 32, 16)] = lo * wev
                tgt[e, pl.ds(jj * 32 + 16, 16)] = hi * wev
            return 0

        lax.fori_loop(0, CEC, _body, 0, unroll=2)

    crow = (sid * NC + cid) * CCH

    def _group(gk, _):
        cbase = crow + gk * GC
        pltpu.sync_copy(src_hbm.at[pl.ds(cbase, GC)], src2)
        pltpu.sync_copy(dst_hbm.at[pl.ds(cbase, GC)], dst2)
        pltpu.sync_copy(norm_hbm.at[pl.ds(cbase, GC)], nrm2)

        def _adj(i, _):
            sl = pl.ds((i % (CEC // 16)) * 16, 16)
            k = i // (CEC // 16)
            dst2b[k, sl] = dst2[k, sl] + NPH
            return 0

        lax.fori_loop(0, GC * (CEC // 16), _adj, 0)

        pltpu.async_copy(y_hbm.at[src2.at[0]], bb0, gs0)

        def _pair(p, _):
            for i in range(2):
                lk = 2 * p + i
                pltpu.make_async_copy(
                    y_hbm.at[src2.at[lk]], bbufs[i], gsems[i]).wait()

                @pl.when(lk + 1 < GC)
                def _():
                    pltpu.async_copy(
                        y_hbm.at[src2.at[lk + 1]], bbufs[1 - i],
                        gsems[1 - i])

                if i == 0:
                    @pl.when((gk > 0) | (p > 0))
                    def _():
                        pltpu.make_async_copy(
                            sbA, acc_sh.at[dst2.at[GC - 1]], ssA).wait()
                        pltpu.make_async_copy(
                            sbB, acc_sh.at[dst2b.at[GC - 1]], ssB).wait()
                else:
                    pltpu.make_async_copy(
                        sbA, acc_sh.at[dst2.at[lk - 1]], ssA).wait()
                    pltpu.make_async_copy(
                        sbB, acc_sh.at[dst2b.at[lk - 1]], ssB).wait()
                _scale(bbufs[i], lk)
                pltpu.async_copy(
                    sbA, acc_sh.at[dst2.at[lk]], ssA, add=True)
                pltpu.async_copy(
                    sbB, acc_sh.at[dst2b.at[lk]], ssB, add=True)
            return 0

        lax.fori_loop(0, GC // 2, _pair, 0)
        return 0

    lax.fori_loop(0, CCH // GC, _group, 0)
    pltpu.make_async_copy(sbA, acc_sh.at[dst2.at[GC - 1]], ssA).wait()
    pltpu.make_async_copy(sbB, acc_sh.at[dst2b.at[GC - 1]], ssB).wait()
    plsc.subcore_barrier()

    def _wb(b, _):
        lbase = sid * RW + b * CEC
        pltpu.sync_copy(acc_sh.at[pl.ds(lbase, CEC)], sbA)
        pltpu.sync_copy(acc_sh.at[pl.ds(NPH + lbase, CEC)], sbB)
        pltpu.sync_copy(y_hbm.at[pl.ds(nbase + b * CEC, CEC)], bb0)

        def _fin(r, _):
            dv2 = plsc.load_gather(dinv2_v, [izero + b * CEC + r])
            for j in range(D // 32):
                yab = plsc.bitcast(bb0[r, pl.ds(j * 16, 16)], _bf16)
                ylo, yhi = plsc.unpack(yab, format=plsc.PackFormat.INTERLEAVED)
                tgt = sbA if j < 4 else sbB
                jj = j % 4
                sl0 = pl.ds(jj * 32, 16)
                sl1 = pl.ds(jj * 32 + 16, 16)
                tgt[r, sl0] = tgt[r, sl0] + ylo * dv2
                tgt[r, sl1] = tgt[r, sl1] + yhi * dv2
            return 0

        lax.fori_loop(0, CEC, _fin, 0)
        pltpu.sync_copy(sbA, acc_hbm.at[pl.ds(nbase + b * CEC, CEC)])
        pltpu.sync_copy(sbB, acc_hbm.at[pl.ds(NP + nbase + b * CEC, CEC)])
        return 0

    lax.fori_loop(0, RW // CEC, _wb, 0)


def _dinv_body(deg_ref, dinv_ref, dinv2_ref):
    deg = jnp.sum(deg_ref[...], axis=0) + 1.0
    dinv = lax.rsqrt(deg)
    dinv_ref[...] = dinv
    dinv2_ref[...] = dinv * dinv


def _dinv(deg32):
    return pl.pallas_call(
        _dinv_body,
        out_shape=[jax.ShapeDtypeStruct((NP,), _f32),
                   jax.ShapeDtypeStruct((NP,), _f32)],
    )(deg32)


def _mm0_body(x_ref, w_ref, y_ref):
    xw = jnp.dot(x_ref[...], w_ref[...], preferred_element_type=_f32)
    y_ref[...] = xw.astype(_bf16)


def _mm0(x, W):
    return pl.pallas_call(
        _mm0_body,
        grid=(NP // RB,),
        in_specs=[pl.BlockSpec((RB, D), lambda i: (i, 0)),
                  pl.BlockSpec((D, D), lambda i: (0, 0))],
        out_specs=pl.BlockSpec((RB, D), lambda i: (i, 0)),
        out_shape=jax.ShapeDtypeStruct((NP, D), _bf16),
    )(x, W)


def _layer_body(x_ref, a0_ref, a1_ref, b_ref, w_ref, xn_ref, y_ref):
    acc = jnp.concatenate([a0_ref[...], a1_ref[...]], axis=1)
    xn = jnp.maximum(x_ref[...] + acc + b_ref[...], 0.0)
    xn_ref[...] = xn
    xw = jnp.dot(xn, w_ref[...], preferred_element_type=_f32)
    y_ref[...] = xw.astype(_bf16)


def _layer(x, acc, bvec, W):
    return pl.pallas_call(
        _layer_body,
        grid=(NP // RB,),
        in_specs=[pl.BlockSpec((RB, D), lambda i: (i, 0)),
                  pl.BlockSpec((RB, H2), lambda i: (i, 0)),
                  pl.BlockSpec((RB, H2), lambda i: (NP // RB + i, 0)),
                  pl.BlockSpec((1, D), lambda i: (0, 0)),
                  pl.BlockSpec((D, D), lambda i: (0, 0))],
        out_specs=[pl.BlockSpec((RB, D), lambda i: (i, 0)),
                   pl.BlockSpec((RB, D), lambda i: (i, 0))],
        out_shape=[jax.ShapeDtypeStruct((NP, D), _f32),
                   jax.ShapeDtypeStruct((NP, D), _bf16)],
    )(x, acc, acc, bvec, W)


def _comb_body(x_ref, a0_ref, a1_ref, b_ref, xn_ref):
    acc = jnp.concatenate([a0_ref[...], a1_ref[...]], axis=1)
    xn_ref[...] = jnp.maximum(x_ref[...] + acc + b_ref[...], 0.0)


def _comb(x, acc, bvec):
    return pl.pallas_call(
        _comb_body,
        grid=(NP // RB,),
        in_specs=[pl.BlockSpec((RB, D), lambda i: (i, 0)),
                  pl.BlockSpec((RB, H2), lambda i: (i, 0)),
                  pl.BlockSpec((RB, H2), lambda i: (NP // RB + i, 0)),
                  pl.BlockSpec((1, D), lambda i: (0, 0))],
        out_specs=pl.BlockSpec((RB, D), lambda i: (i, 0)),
        out_shape=jax.ShapeDtypeStruct((NP, D), _f32),
    )(x, acc, acc, bvec)


def kernel(h_headline, h_para, edge_index, edge_weight, W0, b0, W1, b1, W2, b2):
    x = jnp.concatenate([h_headline[:, None, :], h_para], axis=1)
    x = x.reshape(N, D)
    x = jnp.pad(x, ((0, NP - N), (0, 0)))
    npad = EP - E
    src = jnp.concatenate(
        [edge_index[0].astype(jnp.int32), jnp.zeros((npad,), jnp.int32)])
    dst = jnp.concatenate(
        [edge_index[1].astype(jnp.int32),
         (jnp.arange(npad, dtype=jnp.int32) % 2) * NPH])
    w = jnp.pad(edge_weight.astype(_f32), (0, npad))

    deg32 = _deg_kernel(dst, w)
    dinv, dinv2 = _dinv(deg32)
    srcp, dstp, nrmp = _prep_kernel(src, dst, w, dinv)

    src2 = srcp.reshape(NS * NC * CCH, CEC)
    dst2 = dstp.reshape(NS * NC * CCH, CEC)
    nrm2 = nrmp.reshape(NS * NC * CCH, CEC)

    def _to_i32(yb):
        return lax.bitcast_convert_type(
            yb.reshape(NP, H2, 2), jnp.int32)

    Ws = [jnp.take(Wi, _COLPERM, axis=1) for Wi in (W0, W1, W2)]
    bs = [b0, b1, b2]
    y = _to_i32(_mm0(x, Ws[0]))
    outs = []
    for i in range(L):
        acc = _conv_kernel(y, src2, dst2, nrm2, dinv2)
        if i < L - 1:
            x, y3 = _layer(x, acc, bs[i].reshape(1, D), Ws[i + 1])
            y = _to_i32(y3)
        else:
            x = _comb(x, acc, bs[i].reshape(1, D))
        outs.append(x)

    out = jnp.concatenate(outs, axis=-1)[:N].reshape(B, P + 1, L * D)
    return (out[:, :1, :], out[:, 1:, :])


# speedup vs baseline: 1.0133x; 1.0133x over previous
import functools

import jax
import jax.numpy as jnp
import numpy as np
from jax import lax
from jax.experimental import pallas as pl
from jax.experimental.pallas import tpu as pltpu
from jax.experimental.pallas import tpu_sc as plsc

B, P, D, E, L = 100, 99, 256, 160000, 3
N = B * (P + 1)
NP = 10240
NPH = NP // 2
EP = 163840
NC, NS = 2, 16
ET = EP // NS
EA = EP // (NC * NS)
CAP = 5632
CEC = 32
CCH = CAP // CEC
GC = 16
RW = NPH // NS
H2 = D // 2
RB = 1280

_mesh = plsc.VectorSubcoreMesh(
    core_axis_name="c", subcore_axis_name="s", num_cores=NC, num_subcores=NS)

_f32 = jnp.float32
_bf16 = jnp.bfloat16

_cp = np.empty((D,), np.int32)
for _j in range(D // 32):
    for _t in range(16):
        _cp[32 * _j + 2 * _t] = 32 * _j + _t
        _cp[32 * _j + 2 * _t + 1] = 32 * _j + 16 + _t
_COLPERM = _cp


@functools.partial(
    pl.kernel,
    out_type=jax.ShapeDtypeStruct((NC * NS, NP), _f32),
    mesh=_mesh,
    compiler_params=pltpu.CompilerParams(needs_layout_passes=False),
    scratch_types=[
        pltpu.VMEM((NP,), _f32),
        pltpu.VMEM((EA,), jnp.int32),
        pltpu.VMEM((EA,), _f32),
    ],
)
def _deg_kernel(dst_hbm, w_hbm, deg_hbm, acc_v, dstv, wv):
    cid = lax.axis_index("c")
    sid = lax.axis_index("s")
    wid = cid * NS + sid
    zero16 = lax.iota(jnp.int32, 16).astype(_f32) * 0.0

    def _z(i, _):
        acc_v[pl.ds(i * 16, 16)] = zero16
        return 0

    lax.fori_loop(0, NP // 16, _z, 0)
    base = wid * EA
    pltpu.sync_copy(dst_hbm.at[pl.ds(base, EA)], dstv)
    pltpu.sync_copy(w_hbm.at[pl.ds(base, EA)], wv)

    def _grp(g, _):
        sl = pl.ds(g * 16, 16)
        plsc.addupdate_scatter(acc_v, [dstv[sl]], wv[sl])
        return 0

    lax.fori_loop(0, EA // 16, _grp, 0)
    pltpu.sync_copy(acc_v, deg_hbm.at[wid])


@functools.partial(
    pl.kernel,
    out_type=(jax.ShapeDtypeStruct((NS * NC, CAP), jnp.int32),
              jax.ShapeDtypeStruct((NS * NC, CAP), jnp.int32),
              jax.ShapeDtypeStruct((NS * NC, CAP), _f32)),
    mesh=_mesh,
    compiler_params=pltpu.CompilerParams(needs_layout_passes=False),
    scratch_types=[
        pltpu.VMEM((NP,), _f32),
        pltpu.VMEM((ET,), jnp.int32),
        pltpu.VMEM((ET,), jnp.int32),
        pltpu.VMEM((ET,), _f32),
        pltpu.VMEM((CAP + 16,), jnp.int32),
        pltpu.VMEM((CAP + 16,), jnp.int32),
        pltpu.VMEM((CAP + 16,), _f32),
        pltpu.VMEM((CAP + 16,), jnp.int32),
        pltpu.VMEM((CAP + 16,), jnp.int32),
        pltpu.VMEM((CAP + 16,), _f32),
    ],
)
def _prep_kernel(src_hbm, dst_hbm, w_hbm, dinv_hbm,
                 srcp_hbm, dstp_hbm, nrmp_hbm,
                 dinv_v, srcv, dstv, wv, sA, dA, nA, sB, dB, nB):
    cid = lax.axis_index("c")
    sid = lax.axis_index("s")

    @pl.when(cid == 0)
    def _():
        izero = lax.iota(jnp.int32, 16) * 0
        zero16 = izero.astype(_f32)
        pltpu.sync_copy(dinv_hbm, dinv_v)
        base = sid * ET
        pltpu.sync_copy(src_hbm.at[pl.ds(base, ET)], srcv)
        pltpu.sync_copy(dst_hbm.at[pl.ds(base, ET)], dstv)
        pltpu.sync_copy(w_hbm.at[pl.ds(base, ET)], wv)

        def _z(i, _):
            sl = pl.ds(i * 16, 16)
            sA[sl] = izero
            dA[sl] = izero
            nA[sl] = zero16
            sB[sl] = izero
            dB[sl] = izero
            nB[sl] = zero16
            return 0

        lax.fori_loop(0, (CAP + 16) // 16, _z, 0)

        def _g(g, carry):
            cA, cB = carry
            sl = pl.ds(g * 16, 16)
            s16 = srcv[sl]
            d16 = dstv[sl]
            nrm = (wv[sl] * plsc.load_gather(dinv_v, [s16])
                   * plsc.load_gather(dinv_v, [d16]))
            m0 = d16 < NPH
            m1 = jnp.logical_not(m0)
            pc0 = plsc.cumsum(m0.astype(jnp.int32))
            pc1 = plsc.cumsum(m1.astype(jnp.int32))
            pos0 = cA + pc0 - 1
            pos1 = cB + pc1 - 1
            plsc.store_scatter(sA, [pos0], s16, mask=m0)
            plsc.store_scatter(dA, [pos0], d16, mask=m0)
            plsc.store_scatter(nA, [pos0], nrm, mask=m0)
            plsc.store_scatter(sB, [pos1], s16, mask=m1)
            plsc.store_scatter(dB, [pos1], d16 - NPH, mask=m1)
            plsc.store_scatter(nB, [pos1], nrm, mask=m1)
            k0 = plsc.all_reduce_population_count(m0)[0]
            return (cA + k0, cB + (16 - k0))

        lax.fori_loop(0, ET // 16, _g, (0, 0))
        pltpu.sync_copy(sA.at[pl.ds(0, CAP)], srcp_hbm.at[sid * NC])
        pltpu.sync_copy(dA.at[pl.ds(0, CAP)], dstp_hbm.at[sid * NC])
        pltpu.sync_copy(nA.at[pl.ds(0, CAP)], nrmp_hbm.at[sid * NC])
        pltpu.sync_copy(sB.at[pl.ds(0, CAP)], srcp_hbm.at[sid * NC + 1])
        pltpu.sync_copy(dB.at[pl.ds(0, CAP)], dstp_hbm.at[sid * NC + 1])
        pltpu.sync_copy(nB.at[pl.ds(0, CAP)], nrmp_hbm.at[sid * NC + 1])


@functools.partial(
    pl.kernel,
    out_type=jax.ShapeDtypeStruct((2 * NP, H2), _f32),
    mesh=_mesh,
    compiler_params=pltpu.CompilerParams(needs_layout_passes=False),
    scratch_types=[
        pltpu.VMEM((RW,), _f32),
        pltpu.VMEM((GC, CEC), jnp.int32),
        pltpu.VMEM((GC, CEC), jnp.int32),
        pltpu.VMEM((GC, CEC), jnp.int32),
        pltpu.VMEM((GC, CEC), _f32),
        pltpu.VMEM((CEC, H2), jnp.int32),
        pltpu.VMEM((CEC, H2), jnp.int32),
        pltpu.VMEM((CEC, H2), _f32),
        pltpu.VMEM((CEC, H2), _f32),
        pltpu.VMEM((CEC, H2), _f32),
        pltpu.VMEM((CEC, H2), _f32),
        pltpu.VMEM_SHARED((2 * NPH, H2), _f32),
        pltpu.SemaphoreType.DMA,
        pltpu.SemaphoreType.DMA,
        pltpu.SemaphoreType.DMA,
        pltpu.SemaphoreType.DMA,
        pltpu.SemaphoreType.DMA,
        pltpu.SemaphoreType.DMA,
    ],
)
def _conv_kernel(y_hbm, src_hbm, dst_hbm, norm_hbm, dinv2_hbm, acc_hbm,
                 dinv2_v, src2, dst2, dst2b, nrm2, bb0, bb1,
                 sbA0, sbA1, sbB0, sbB1,
                 acc_sh, gs0, gs1, ssA0, ssA1, ssB0, ssB1):
    cid = lax.axis_index("c")
    sid = lax.axis_index("s")
    izero = lax.iota(jnp.int32, 16) * 0
    bbufs = [bb0, bb1]
    gsems = [gs0, gs1]
    sbAs = [sbA0, sbA1]
    sbBs = [sbB0, sbB1]
    ssAs = [ssA0, ssA1]
    ssBs = [ssB0, ssB1]
    nbase = cid * NPH + sid * RW

    pltpu.sync_copy(dinv2_hbm.at[pl.ds(nbase, RW)], dinv2_v)

    zero16 = izero.astype(_f32)

    def _zv(i, _):
        sbA0[i // (H2 // 16), pl.ds((i % (H2 // 16)) * 16, 16)] = zero16
        return 0

    lax.fori_loop(0, CEC * (H2 // 16), _zv, 0)
    for b in range(RW // CEC):
        pltpu.sync_copy(sbA0, acc_sh.at[pl.ds(sid * RW + b * CEC, CEC)])
        pltpu.sync_copy(sbA0, acc_sh.at[pl.ds(NPH + sid * RW + b * CEC, CEC)])
    plsc.subcore_barrier()

    def _scale(bbuf, sbA, sbB, k):
        def _body(e, _):
            wev = plsc.load_gather(nrm2.at[k], [izero + e])
            for j in range(D // 32):
                ab = plsc.bitcast(bbuf[e, pl.ds(j * 16, 16)], _bf16)
                lo, hi = plsc.unpack(ab, format=plsc.PackFormat.INTERLEAVED)
                tgt = sbA if j < 4 else sbB
                jj = j % 4
                tgt[e, pl.ds(jj * 32, 16)] = lo * wev
                tgt[e, pl.ds(jj * 32 + 16, 16)] = hi * wev
            return 0

        lax.fori_loop(0, CEC, _body, 0, unroll=2)

    crow = (sid * NC + cid) * CCH

    def _group(gk, _):
        cbase = crow + gk * GC
        pltpu.sync_copy(src_hbm.at[pl.ds(cbase, GC)], src2)
        pltpu.sync_copy(dst_hbm.at[pl.ds(cbase, GC)], dst2)
        pltpu.sync_copy(norm_hbm.at[pl.ds(cbase, GC)], nrm2)

        def _adj(i, _):
            sl = pl.ds((i % (CEC // 16)) * 16, 16)
            k = i // (CEC // 16)
            dst2b[k, sl] = dst2[k, sl] + NPH
            return 0

        lax.fori_loop(0, GC * (CEC // 16), _adj, 0)

        pltpu.async_copy(y_hbm.at[src2.at[0]], bb0, gs0)

        def _pair(p, _):
            for i in range(2):
                lk = 2 * p + i
                pltpu.make_async_copy(
                    y_hbm.at[src2.at[lk]], bbufs[i], gsems[i]).wait()

                @pl.when(lk + 1 < GC)
                def _():
                    pltpu.async_copy(
                        y_hbm.at[src2.at[lk + 1]], bbufs[1 - i],
                        gsems[1 - i])

                @pl.when((gk > 0) | (p > 0))
                def _():
                    pltpu.make_async_copy(
                        sbAs[i], acc_sh.at[dst2.at[GC - 1]], ssAs[i]).wait()
                    pltpu.make_async_copy(
                        sbBs[i], acc_sh.at[dst2b.at[GC - 1]], ssBs[i]).wait()
                _scale(bbufs[i], sbAs[i], sbBs[i], lk)
                pltpu.async_copy(
                    sbAs[i], acc_sh.at[dst2.at[lk]], ssAs[i], add=True)
                pltpu.async_copy(
                    sbBs[i], acc_sh.at[dst2b.at[lk]], ssBs[i], add=True)
            return 0

        lax.fori_loop(0, GC // 2, _pair, 0)
        return 0

    lax.fori_loop(0, CCH // GC, _group, 0)
    for i in range(2):
        pltpu.make_async_copy(
            sbAs[i], acc_sh.at[dst2.at[GC - 1]], ssAs[i]).wait()
        pltpu.make_async_copy(
            sbBs[i], acc_sh.at[dst2b.at[GC - 1]], ssBs[i]).wait()
    plsc.subcore_barrier()

    def _wb(b, _):
        lbase = sid * RW + b * CEC
        pltpu.sync_copy(acc_sh.at[pl.ds(lbase, CEC)], sbA0)
        pltpu.sync_copy(acc_sh.at[pl.ds(NPH + lbase, CEC)], sbB0)
        pltpu.sync_copy(y_hbm.at[pl.ds(nbase + b * CEC, CEC)], bb0)

        def _fin(r, _):
            dv2 = plsc.load_gather(dinv2_v, [izero + b * CEC + r])
            for j in range(D // 32):
                yab = plsc.bitcast(bb0[r, pl.ds(j * 16, 16)], _bf16)
                ylo, yhi = plsc.unpack(yab, format=plsc.PackFormat.INTERLEAVED)
                tgt = sbA0 if j < 4 else sbB0
                jj = j % 4
                sl0 = pl.ds(jj * 32, 16)
                sl1 = pl.ds(jj * 32 + 16, 16)
                tgt[r, sl0] = tgt[r, sl0] + ylo * dv2
                tgt[r, sl1] = tgt[r, sl1] + yhi * dv2
            return 0

        lax.fori_loop(0, CEC, _fin, 0)
        pltpu.sync_copy(sbA0, acc_hbm.at[pl.ds(nbase + b * CEC, CEC)])
        pltpu.sync_copy(sbB0, acc_hbm.at[pl.ds(NP + nbase + b * CEC, CEC)])
        return 0

    lax.fori_loop(0, RW // CEC, _wb, 0)


def _dinv_body(deg_ref, dinv_ref, dinv2_ref):
    deg = jnp.sum(deg_ref[...], axis=0) + 1.0
    dinv = lax.rsqrt(deg)
    dinv_ref[...] = dinv
    dinv2_ref[...] = dinv * dinv


def _dinv(deg32):
    return pl.pallas_call(
        _dinv_body,
        out_shape=[jax.ShapeDtypeStruct((NP,), _f32),
                   jax.ShapeDtypeStruct((NP,), _f32)],
    )(deg32)


def _mm0_body(x_ref, w_ref, y_ref):
    xw = jnp.dot(x_ref[...], w_ref[...], preferred_element_type=_f32)
    y_ref[...] = xw.astype(_bf16)


def _mm0(x, W):
    return pl.pallas_call(
        _mm0_body,
        grid=(NP // RB,),
        in_specs=[pl.BlockSpec((RB, D), lambda i: (i, 0)),
                  pl.BlockSpec((D, D), lambda i: (0, 0))],
        out_specs=pl.BlockSpec((RB, D), lambda i: (i, 0)),
        out_shape=jax.ShapeDtypeStruct((NP, D), _bf16),
    )(x, W)


def _layer_body(x_ref, a0_ref, a1_ref, b_ref, w_ref, xn_ref, y_ref):
    acc = jnp.concatenate([a0_ref[...], a1_ref[...]], axis=1)
    xn = jnp.maximum(x_ref[...] + acc + b_ref[...], 0.0)
    xn_ref[...] = xn
    xw = jnp.dot(xn, w_ref[...], preferred_element_type=_f32)
    y_ref[...] = xw.astype(_bf16)


def _layer(x, acc, bvec, W):
    return pl.pallas_call(
        _layer_body,
        grid=(NP // RB,),
        in_specs=[pl.BlockSpec((RB, D), lambda i: (i, 0)),
                  pl.BlockSpec((RB, H2), lambda i: (i, 0)),
                  pl.BlockSpec((RB, H2), lambda i: (NP // RB + i, 0)),
                  pl.BlockSpec((1, D), lambda i: (0, 0)),
                  pl.BlockSpec((D, D), lambda i: (0, 0))],
        out_specs=[pl.BlockSpec((RB, D), lambda i: (i, 0)),
                   pl.BlockSpec((RB, D), lambda i: (i, 0))],
        out_shape=[jax.ShapeDtypeStruct((NP, D), _f32),
                   jax.ShapeDtypeStruct((NP, D), _bf16)],
    )(x, acc, acc, bvec, W)


def _comb_body(x_ref, a0_ref, a1_ref, b_ref, xn_ref):
    acc = jnp.concatenate([a0_ref[...], a1_ref[...]], axis=1)
    xn_ref[...] = jnp.maximum(x_ref[...] + acc + b_ref[...], 0.0)


def _comb(x, acc, bvec):
    return pl.pallas_call(
        _comb_body,
        grid=(NP // RB,),
        in_specs=[pl.BlockSpec((RB, D), lambda i: (i, 0)),
                  pl.BlockSpec((RB, H2), lambda i: (i, 0)),
                  pl.BlockSpec((RB, H2), lambda i: (NP // RB + i, 0)),
                  pl.BlockSpec((1, D), lambda i: (0, 0))],
        out_specs=pl.BlockSpec((RB, D), lambda i: (i, 0)),
        out_shape=jax.ShapeDtypeStruct((NP, D), _f32),
    )(x, acc, acc, bvec)


def kernel(h_headline, h_para, edge_index, edge_weight, W0, b0, W1, b1, W2, b2):
    x = jnp.concatenate([h_headline[:, None, :], h_para], axis=1)
    x = x.reshape(N, D)
    x = jnp.pad(x, ((0, NP - N), (0, 0)))
    npad = EP - E
    src = jnp.concatenate(
        [edge_index[0].astype(jnp.int32), jnp.zeros((npad,), jnp.int32)])
    dst = jnp.concatenate(
        [edge_index[1].astype(jnp.int32),
         (jnp.arange(npad, dtype=jnp.int32) % 2) * NPH])
    w = jnp.pad(edge_weight.astype(_f32), (0, npad))

    deg32 = _deg_kernel(dst, w)
    dinv, dinv2 = _dinv(deg32)
    srcp, dstp, nrmp = _prep_kernel(src, dst, w, dinv)

    src2 = srcp.reshape(NS * NC * CCH, CEC)
    dst2 = dstp.reshape(NS * NC * CCH, CEC)
    nrm2 = nrmp.reshape(NS * NC * CCH, CEC)

    def _to_i32(yb):
        return lax.bitcast_convert_type(
            yb.reshape(NP, H2, 2), jnp.int32)

    Ws = [jnp.take(Wi, _COLPERM, axis=1) for Wi in (W0, W1, W2)]
    bs = [b0, b1, b2]
    y = _to_i32(_mm0(x, Ws[0]))
    outs = []
    for i in range(L):
        acc = _conv_kernel(y, src2, dst2, nrm2, dinv2)
        if i < L - 1:
            x, y3 = _layer(x, acc, bs[i].reshape(1, D), Ws[i + 1])
            y = _to_i32(y3)
        else:
            x = _comb(x, acc, bs[i].reshape(1, D))
        outs.append(x)

    out = jnp.concatenate(outs, axis=-1)[:N].reshape(B, P + 1, L * D)
    return (out[:, :1, :], out[:, 1:, :])
